# Initial kernel scaffold; baseline (speedup 1.0000x reference)
import jax
import jax.numpy as jnp
from jax.experimental import pallas as pl


def kernel(x, edge_index, W1, b1, W2, b2, W3, b3, Wg, bg):
    raise NotImplementedError("write your pallas kernel here")


# speedup vs baseline: 4.5452x; 4.5452x over previous
import functools

import jax
import jax.numpy as jnp
from jax import lax
from jax.experimental import pallas as pl
from jax.experimental.pallas import tpu as pltpu
from jax.experimental.pallas import tpu_sc as plsc

N = 10000
NP = 10240
E = 320000
NC = 2
NS = 16
CH = 80
L = 16
DR = NP // 128
BM = 1000


@functools.lru_cache(maxsize=None)
def _mesh():
  return plsc.VectorSubcoreMesh(
      core_axis_name="c", subcore_axis_name="s", num_cores=NC, num_subcores=NS)


def _edge_loop(table_hbm, src_hbm, dst_hbm, acc, src_v, dst_v, rows_v, sem,
               base, n_chunks, deg_local=None):
  ones16 = jnp.ones((L,), jnp.float32)

  def body(i, carry):
    off = pl.multiple_of(base + i * CH, 8)
    pltpu.sync_copy(src_hbm.at[pl.ds(off, CH)], src_v)
    pltpu.sync_copy(dst_hbm.at[pl.ds(off, CH)], dst_v)
    gather = pltpu.async_copy(table_hbm.at[src_v], rows_v, sem)
    if deg_local is not None:
      for j in range(CH // L):
        d16 = dst_v[pl.ds(j * L, L)]
        plsc.addupdate_scatter(deg_local, [d16], ones16)
    gather.wait()
    pltpu.sync_copy(rows_v, acc.at[dst_v], add=True)
    return carry

  lax.fori_loop(0, n_chunks, body, 0)


@functools.lru_cache(maxsize=None)
def _make_agg_edge_split(with_deg):
  ncols = 128
  ept = E // (NC * NS)
  n_chunks = ept // CH
  rpt = NP // NS

  out_type = jax.ShapeDtypeStruct((NC, NP, ncols), jnp.float32)
  if with_deg:
    out_type = [out_type, jax.ShapeDtypeStruct((NC * NS, NP), jnp.float32)]
  scratch = [
      pltpu.VMEM((CH,), jnp.int32),
      pltpu.VMEM((CH,), jnp.int32),
      pltpu.VMEM((CH, ncols), jnp.float32),
      pltpu.VMEM_SHARED((NP, ncols), jnp.float32),
      pltpu.SemaphoreType.DMA,
  ]
  if with_deg:
    scratch.append(pltpu.VMEM((NP,), jnp.float32))

  @functools.partial(pl.kernel, out_type=out_type, mesh=_mesh(),
                     scratch_types=scratch,
                     compiler_params=pltpu.CompilerParams(
                         needs_layout_passes=False))
  def agg(table_hbm, src_hbm, dst_hbm, zero_hbm, *refs):
    if with_deg:
      out_hbm, deg_hbm, src_v, dst_v, rows_v, acc, sem, deg_local = refs
    else:
      out_hbm, src_v, dst_v, rows_v, acc, sem = refs
      deg_local = None
    cid = lax.axis_index("c")
    sid = lax.axis_index("s")
    row0 = pl.multiple_of(sid * rpt, 8)
    pltpu.sync_copy(zero_hbm, acc.at[pl.ds(row0, rpt)])
    if with_deg:
      z16 = jnp.zeros((L,), jnp.float32)

      def zbody(i, carry):
        deg_local[pl.ds(pl.multiple_of(i * L, 8), L)] = z16
        return carry

      lax.fori_loop(0, NP // L, zbody, 0)
    plsc.subcore_barrier()
    base = (cid * NS + sid) * ept
    _edge_loop(table_hbm, src_hbm, dst_hbm, acc, src_v, dst_v, rows_v, sem,
               base, n_chunks, deg_local)
    plsc.subcore_barrier()
    pltpu.sync_copy(acc.at[pl.ds(row0, rpt)], out_hbm.at[cid, pl.ds(row0, rpt)])
    if with_deg:
      pltpu.sync_copy(deg_local, deg_hbm.at[cid * NS + sid])

  return agg


@functools.lru_cache(maxsize=None)
def _make_agg_feat_split():
  ncols = 128
  ept = E // NS
  n_chunks = ept // CH
  rpt = NP // NS

  @functools.partial(
      pl.kernel,
      out_type=jax.ShapeDtypeStruct((NC, NP, ncols), jnp.float32),
      mesh=_mesh(),
      scratch_types=[
          pltpu.VMEM((CH,), jnp.int32),
          pltpu.VMEM((CH,), jnp.int32),
          pltpu.VMEM((CH, ncols), jnp.float32),
          pltpu.VMEM_SHARED((NP, ncols), jnp.float32),
          pltpu.SemaphoreType.DMA,
      ])
  def agg(tables_hbm, src_hbm, dst_hbm, zero_hbm, out_hbm,
          src_v, dst_v, rows_v, acc, sem):
    cid = lax.axis_index("c")
    sid = lax.axis_index("s")
    row0 = pl.multiple_of(sid * rpt, 8)
    pltpu.sync_copy(zero_hbm, acc.at[pl.ds(row0, rpt)])
    plsc.subcore_barrier()
    base = sid * ept

    @pl.when(cid == 0)
    def _():
      _edge_loop(tables_hbm.at[0], src_hbm, dst_hbm, acc, src_v, dst_v,
                 rows_v, sem, base, n_chunks)

    @pl.when(cid == 1)
    def _():
      _edge_loop(tables_hbm.at[1], src_hbm, dst_hbm, acc, src_v, dst_v,
                 rows_v, sem, base, n_chunks)

    plsc.subcore_barrier()
    pltpu.sync_copy(acc.at[pl.ds(row0, rpt)], out_hbm.at[cid, pl.ds(row0, rpt)])

  return agg


_DOT = functools.partial(jnp.dot, preferred_element_type=jnp.float32,
                         precision=jax.lax.Precision.HIGHEST)


def _invdeg_body(dp_ref, inv_ref):
  deg = jnp.sum(dp_ref[...], axis=0)
  inv_ref[...] = 1.0 / jnp.maximum(deg, 1.0)


def _layer1_body(p_ref, d_ref, w_ref, b_ref, h_ref):
  a = (p_ref[0] + p_ref[1]) * d_ref[...]
  h_ref[...] = jnp.maximum(_DOT(a, w_ref[...]) + b_ref[...], 0.0)


def _layer2_body(p_ref, d_ref, w_ref, b_ref, h_ref):
  a = (p_ref[0] + p_ref[1]) * d_ref[...]
  h = jnp.maximum(_DOT(a, w_ref[...]) + b_ref[...], 0.0)
  h_ref[0] = h[:, :128]
  h_ref[1] = h[:, 128:]


def _layer3_body(p_ref, d_ref, w3_ref, b3_ref, wg_ref, bg_ref, out_ref,
                 acc_ref):
  i = pl.program_id(0)

  @pl.when(i == 0)
  def _():
    acc_ref[...] = jnp.zeros_like(acc_ref)

  a = jnp.concatenate([p_ref[0], p_ref[1]], axis=1) * d_ref[...]
  h = jnp.maximum(_DOT(a, w3_ref[...]) + b3_ref[...], 0.0)
  acc_ref[...] += jnp.sum(h, axis=0, keepdims=True)

  @pl.when(i == pl.num_programs(0) - 1)
  def _():
    out_ref[...] = _DOT(acc_ref[...] / N, wg_ref[...]) + bg_ref[...]


def _full(shape):
  return pl.BlockSpec(shape, lambda i: tuple(0 for _ in shape))


_P_SPEC = pl.BlockSpec((2, BM, 128), lambda i: (0, i, 0))
_D_SPEC = pl.BlockSpec((BM, 1), lambda i: (i, 0))


def kernel(x, edge_index, W1, b1, W2, b2, W3, b3, Wg, bg):
  src = edge_index[0]
  dst = edge_index[1]
  z128 = jnp.zeros((NP // NS, 128), jnp.float32)

  parts1, deg_parts = _make_agg_edge_split(True)(x, src, dst, z128)

  inv = pl.pallas_call(
      _invdeg_body,
      grid=(1,),
      in_specs=[_full((NC * NS, DR, 128))],
      out_specs=_full((DR, 128)),
      out_shape=jax.ShapeDtypeStruct((DR, 128), jnp.float32),
  )(deg_parts.reshape(NC * NS, DR, 128))
  deg2col = inv.reshape(NP, 1)

  grid = (N // BM,)
  h1 = pl.pallas_call(
      _layer1_body,
      grid=grid,
      in_specs=[_P_SPEC, _D_SPEC, _full((128, 128)), _full((1, 128))],
      out_specs=pl.BlockSpec((BM, 128), lambda i: (i, 0)),
      out_shape=jax.ShapeDtypeStruct((N, 128), jnp.float32),
  )(parts1, deg2col, W1, b1.reshape(1, 128))

  parts2 = _make_agg_edge_split(False)(h1, src, dst, z128)

  h2p = pl.pallas_call(
      _layer2_body,
      grid=grid,
      in_specs=[_P_SPEC, _D_SPEC, _full((128, 256)), _full((1, 256))],
      out_specs=_P_SPEC,
      out_shape=jax.ShapeDtypeStruct((2, N, 128), jnp.float32),
  )(parts2, deg2col, W2, b2.reshape(1, 256))

  parts3 = _make_agg_feat_split()(h2p, src, dst, z128)

  g = pl.pallas_call(
      _layer3_body,
      grid=grid,
      in_specs=[_P_SPEC, _D_SPEC, _full((256, 512)), _full((1, 512)),
                _full((512, 2)), _full((1, 2))],
      out_specs=pl.BlockSpec((1, 2), lambda i: (0, 0)),
      out_shape=jax.ShapeDtypeStruct((1, 2), jnp.float32),
      scratch_shapes=[pltpu.VMEM((1, 512), jnp.float32)],
  )(parts3, deg2col, W3, b3.reshape(1, 512), Wg, bg.reshape(1, 2))

  return g.reshape(2)


# speedup vs baseline: 9.1338x; 2.0096x over previous
import functools

import jax
import jax.numpy as jnp
from jax import lax
from jax.experimental import pallas as pl
from jax.experimental.pallas import tpu as pltpu
from jax.experimental.pallas import tpu_sc as plsc

N = 10000
NP = 10240
E = 320000
NC = 2
NS = 16
CH = 80
L = 16
DR = NP // 128
BM = 1000


@functools.lru_cache(maxsize=None)
def _mesh():
  return plsc.VectorSubcoreMesh(
      core_axis_name="c", subcore_axis_name="s", num_cores=NC, num_subcores=NS)


def _edge_loop(table_hbm, src_hbm, dst_hbm, acc, src_r, dst_r, rows,
               ssems, dsems, gsems, base, n_chunks, U, deg_local=None):
  ones16 = jnp.ones((L,), jnp.float32)

  def _idx_start(c, u):
    off = pl.multiple_of(base + c * CH, 8)
    pltpu.async_copy(src_hbm.at[pl.ds(off, CH)], src_r[u], ssems[u])
    pltpu.async_copy(dst_hbm.at[pl.ds(off, CH)], dst_r[u], dsems[u])

  def _src_wait(u):
    pltpu.make_async_copy(src_hbm.at[pl.ds(0, CH)], src_r[u], ssems[u]).wait()

  def _gather_start(u):
    pltpu.async_copy(table_hbm.at[src_r[u]], rows[u], gsems[u])

  def _consume(u):
    pltpu.make_async_copy(table_hbm.at[pl.ds(0, CH)], rows[u], gsems[u]).wait()
    pltpu.make_async_copy(dst_hbm.at[pl.ds(0, CH)], dst_r[u], dsems[u]).wait()
    pltpu.sync_copy(rows[u], acc.at[dst_r[u]], add=True)
    if deg_local is not None:
      for j in range(CH // L):
        d16 = dst_r[u][pl.ds(j * L, L)]
        plsc.addupdate_scatter(deg_local, [d16], ones16)

  for u in range(U):
    _idx_start(u, u)
  for u in range(U):
    _src_wait(u)
    _gather_start(u)

  def body(m, carry):
    for u in range(U):
      k = m * U + u
      _consume(u)

      @pl.when(k + U < n_chunks)
      def _():
        _idx_start(k + U, u)

    for u in range(U):
      k2 = m * U + U + u

      @pl.when(k2 < n_chunks)
      def _():
        _src_wait(u)
        _gather_start(u)

    return carry

  lax.fori_loop(0, n_chunks // U, body, 0)
  for u in range(n_chunks % U):
    _consume(u)


@functools.lru_cache(maxsize=None)
def _make_agg_edge_split(with_deg):
  ncols = 128
  ept = E // (NC * NS)
  n_chunks = ept // CH
  rpt = NP // NS

  U = 3 if with_deg else 4
  out_type = jax.ShapeDtypeStruct((NC, NP, ncols), jnp.float32)
  if with_deg:
    out_type = [out_type, jax.ShapeDtypeStruct((NC * NS, NP), jnp.float32)]
  scratch = [
      [pltpu.VMEM((CH,), jnp.int32) for _ in range(U)],
      [pltpu.VMEM((CH,), jnp.int32) for _ in range(U)],
      [pltpu.VMEM((CH, ncols), jnp.float32) for _ in range(U)],
      [pltpu.SemaphoreType.DMA for _ in range(U)],
      [pltpu.SemaphoreType.DMA for _ in range(U)],
      [pltpu.SemaphoreType.DMA for _ in range(U)],
      pltpu.VMEM_SHARED((NP, ncols), jnp.float32),
  ]
  if with_deg:
    scratch.append(pltpu.VMEM((NP,), jnp.float32))

  @functools.partial(pl.kernel, out_type=out_type, mesh=_mesh(),
                     scratch_types=scratch,
                     compiler_params=pltpu.CompilerParams(
                         needs_layout_passes=False))
  def agg(table_hbm, src_hbm, dst_hbm, zero_hbm, *refs):
    if with_deg:
      (out_hbm, deg_hbm, src_r, dst_r, rows, ssems, dsems, gsems, acc,
       deg_local) = refs
    else:
      out_hbm, src_r, dst_r, rows, ssems, dsems, gsems, acc = refs
      deg_local = None
    cid = lax.axis_index("c")
    sid = lax.axis_index("s")
    row0 = pl.multiple_of(sid * rpt, 8)
    base = pl.multiple_of((cid * NS + sid) * ept, 8)
    pltpu.sync_copy(zero_hbm, acc.at[pl.ds(row0, rpt)])
    if with_deg:
      z16 = jnp.zeros((L,), jnp.float32)

      def zbody(i, carry):
        deg_local[pl.ds(pl.multiple_of(i * L, 8), L)] = z16
        return carry

      lax.fori_loop(0, NP // L, zbody, 0)
    plsc.subcore_barrier()
    _edge_loop(table_hbm, src_hbm, dst_hbm, acc, src_r, dst_r, rows,
               ssems, dsems, gsems, base, n_chunks, U, deg_local)
    plsc.subcore_barrier()
    pltpu.sync_copy(acc.at[pl.ds(row0, rpt)], out_hbm.at[cid, pl.ds(row0, rpt)])
    if with_deg:
      pltpu.sync_copy(deg_local, deg_hbm.at[cid * NS + sid])

  return agg


@functools.lru_cache(maxsize=None)
def _make_agg_feat_split():
  ncols = 128
  ept = E // NS
  n_chunks = ept // CH
  rpt = NP // NS

  U = 4

  @functools.partial(
      pl.kernel,
      out_type=jax.ShapeDtypeStruct((NC, NP, ncols), jnp.float32),
      mesh=_mesh(),
      compiler_params=pltpu.CompilerParams(needs_layout_passes=False),
      scratch_types=[
          [pltpu.VMEM((CH,), jnp.int32) for _ in range(U)],
          [pltpu.VMEM((CH,), jnp.int32) for _ in range(U)],
          [pltpu.VMEM((CH, ncols), jnp.float32) for _ in range(U)],
          [pltpu.SemaphoreType.DMA for _ in range(U)],
          [pltpu.SemaphoreType.DMA for _ in range(U)],
          [pltpu.SemaphoreType.DMA for _ in range(U)],
          pltpu.VMEM_SHARED((NP, ncols), jnp.float32),
      ])
  def agg(tables_hbm, src_hbm, dst_hbm, zero_hbm, out_hbm,
          src_r, dst_r, rows, ssems, dsems, gsems, acc):
    cid = lax.axis_index("c")
    sid = lax.axis_index("s")
    row0 = pl.multiple_of(sid * rpt, 8)
    base = pl.multiple_of(sid * ept, 8)
    pltpu.sync_copy(zero_hbm, acc.at[pl.ds(row0, rpt)])
    plsc.subcore_barrier()

    @pl.when(cid == 0)
    def _():
      _edge_loop(tables_hbm.at[0], src_hbm, dst_hbm, acc, src_r, dst_r, rows,
                 ssems, dsems, gsems, base, n_chunks, U)

    @pl.when(cid == 1)
    def _():
      _edge_loop(tables_hbm.at[1], src_hbm, dst_hbm, acc, src_r, dst_r, rows,
                 ssems, dsems, gsems, base, n_chunks, U)

    plsc.subcore_barrier()
    pltpu.sync_copy(acc.at[pl.ds(row0, rpt)], out_hbm.at[cid, pl.ds(row0, rpt)])

  return agg


_DOT = functools.partial(jnp.dot, preferred_element_type=jnp.float32,
                         precision=jax.lax.Precision.HIGHEST)


def _invdeg_body(dp_ref, inv_ref):
  deg = jnp.sum(dp_ref[...], axis=0)
  inv_ref[...] = 1.0 / jnp.maximum(deg, 1.0)


def _layer1_body(p_ref, d_ref, w_ref, b_ref, h_ref):
  a = (p_ref[0] + p_ref[1]) * d_ref[...]
  h_ref[...] = jnp.maximum(_DOT(a, w_ref[...]) + b_ref[...], 0.0)


def _layer2_body(p_ref, d_ref, w_ref, b_ref, h_ref):
  a = (p_ref[0] + p_ref[1]) * d_ref[...]
  h = jnp.maximum(_DOT(a, w_ref[...]) + b_ref[...], 0.0)
  h_ref[0] = h[:, :128]
  h_ref[1] = h[:, 128:]


def _layer3_body(p_ref, d_ref, w3_ref, b3_ref, wg_ref, bg_ref, out_ref,
                 acc_ref):
  i = pl.program_id(0)

  @pl.when(i == 0)
  def _():
    acc_ref[...] = jnp.zeros_like(acc_ref)

  a = jnp.concatenate([p_ref[0], p_ref[1]], axis=1) * d_ref[...]
  h = jnp.maximum(_DOT(a, w3_ref[...]) + b3_ref[...], 0.0)
  acc_ref[...] += jnp.sum(h, axis=0, keepdims=True)

  @pl.when(i == pl.num_programs(0) - 1)
  def _():
    out_ref[...] = _DOT(acc_ref[...] / N, wg_ref[...]) + bg_ref[...]


def _full(shape):
  return pl.BlockSpec(shape, lambda i: tuple(0 for _ in shape))


_P_SPEC = pl.BlockSpec((2, BM, 128), lambda i: (0, i, 0))
_D_SPEC = pl.BlockSpec((BM, 1), lambda i: (i, 0))


def kernel(x, edge_index, W1, b1, W2, b2, W3, b3, Wg, bg):
  src = edge_index[0]
  dst = edge_index[1]
  z128 = jnp.zeros((NP // NS, 128), jnp.float32)

  parts1, deg_parts = _make_agg_edge_split(True)(x, src, dst, z128)

  inv = pl.pallas_call(
      _invdeg_body,
      grid=(1,),
      in_specs=[_full((NC * NS, DR, 128))],
      out_specs=_full((DR, 128)),
      out_shape=jax.ShapeDtypeStruct((DR, 128), jnp.float32),
  )(deg_parts.reshape(NC * NS, DR, 128))
  deg2col = inv.reshape(NP, 1)

  grid = (N // BM,)
  h1 = pl.pallas_call(
      _layer1_body,
      grid=grid,
      in_specs=[_P_SPEC, _D_SPEC, _full((128, 128)), _full((1, 128))],
      out_specs=pl.BlockSpec((BM, 128), lambda i: (i, 0)),
      out_shape=jax.ShapeDtypeStruct((N, 128), jnp.float32),
  )(parts1, deg2col, W1, b1.reshape(1, 128))

  parts2 = _make_agg_edge_split(False)(h1, src, dst, z128)

  h2p = pl.pallas_call(
      _layer2_body,
      grid=grid,
      in_specs=[_P_SPEC, _D_SPEC, _full((128, 256)), _full((1, 256))],
      out_specs=_P_SPEC,
      out_shape=jax.ShapeDtypeStruct((2, N, 128), jnp.float32),
  )(parts2, deg2col, W2, b2.reshape(1, 256))

  parts3 = _make_agg_feat_split()(h2p, src, dst, z128)

  g = pl.pallas_call(
      _layer3_body,
      grid=grid,
      in_specs=[_P_SPEC, _D_SPEC, _full((256, 512)), _full((1, 512)),
                _full((512, 2)), _full((1, 2))],
      out_specs=pl.BlockSpec((1, 2), lambda i: (0, 0)),
      out_shape=jax.ShapeDtypeStruct((1, 2), jnp.float32),
      scratch_shapes=[pltpu.VMEM((1, 512), jnp.float32)],
  )(parts3, deg2col, W3, b3.reshape(1, 512), Wg, bg.reshape(1, 2))

  return g.reshape(2)


# speedup vs baseline: 11.0509x; 1.2099x over previous
import functools

import jax
import jax.numpy as jnp
from jax import lax
from jax.experimental import pallas as pl
from jax.experimental.pallas import tpu as pltpu
from jax.experimental.pallas import tpu_sc as plsc

N = 10000
NP = 10240
E = 320000
NC = 2
NS = 16
CH = 80
L = 16
DR = NP // 128
BM = 1000


@functools.lru_cache(maxsize=None)
def _mesh():
  return plsc.VectorSubcoreMesh(
      core_axis_name="c", subcore_axis_name="s", num_cores=NC, num_subcores=NS)


def _edge_loop(table_hbm, src_hbm, dst_hbm, acc, src_r, dst_r, rows,
               ssems, dsems, gsems, asems, base, n_chunks, U, deg_local=None):
  ones16 = jnp.ones((L,), jnp.float32)

  def _src_start(c, u):
    off = pl.multiple_of(base + c * CH, 8)
    pltpu.async_copy(src_hbm.at[pl.ds(off, CH)], src_r[u], ssems[u])

  def _dst_start(c, u):
    off = pl.multiple_of(base + c * CH, 8)
    pltpu.async_copy(dst_hbm.at[pl.ds(off, CH)], dst_r[u], dsems[u])

  def _src_wait(u):
    pltpu.make_async_copy(src_hbm.at[pl.ds(0, CH)], src_r[u], ssems[u]).wait()

  def _dst_wait(u):
    pltpu.make_async_copy(dst_hbm.at[pl.ds(0, CH)], dst_r[u], dsems[u]).wait()

  def _gather_start(u):
    pltpu.async_copy(table_hbm.at[src_r[u]], rows[u], gsems[u])

  def _gather_wait(u):
    pltpu.make_async_copy(table_hbm.at[pl.ds(0, CH)], rows[u], gsems[u]).wait()

  def _scatter_wait(u):
    pltpu.make_async_copy(rows[u], acc.at[pl.ds(0, CH)], asems[u]).wait()

  def _hist(u):
    if deg_local is not None:
      for j in range(CH // L):
        d16 = dst_r[u][pl.ds(j * L, L)]
        plsc.addupdate_scatter(deg_local, [d16], ones16)

  for u in range(U):
    _src_start(u, u)
    _dst_start(u, u)
  for u in range(U):
    _src_wait(u)
    _gather_start(u)

  def body(m, carry):
    for u in range(U):
      k = m * U + u
      _gather_wait(u)

      @pl.when(k + U < n_chunks)
      def _():
        _src_start(k + U, u)

      _dst_wait(u)
      pltpu.async_copy(rows[u], acc.at[dst_r[u]], asems[u], add=True)
      _hist(u)

    for u in range(U):
      k2 = m * U + U + u
      _scatter_wait(u)

      @pl.when(k2 < n_chunks)
      def _():
        _dst_start(k2, u)
        _src_wait(u)
        _gather_start(u)

    return carry

  lax.fori_loop(0, n_chunks // U, body, 0)
  for u in range(n_chunks % U):
    _gather_wait(u)
    _dst_wait(u)
    pltpu.sync_copy(rows[u], acc.at[dst_r[u]], add=True)
    _hist(u)


@functools.lru_cache(maxsize=None)
def _make_agg_edge_split(with_deg):
  ncols = 128
  ept = E // (NC * NS)
  n_chunks = ept // CH
  rpt = NP // NS

  U = 3 if with_deg else 4
  out_type = jax.ShapeDtypeStruct((NC, NP, ncols), jnp.float32)
  if with_deg:
    out_type = [out_type, jax.ShapeDtypeStruct((NC * NS, NP), jnp.float32)]
  scratch = [
      [pltpu.VMEM((CH,), jnp.int32) for _ in range(U)],
      [pltpu.VMEM((CH,), jnp.int32) for _ in range(U)],
      [pltpu.VMEM((CH, ncols), jnp.float32) for _ in range(U)],
      [pltpu.SemaphoreType.DMA for _ in range(U)],
      [pltpu.SemaphoreType.DMA for _ in range(U)],
      [pltpu.SemaphoreType.DMA for _ in range(U)],
      [pltpu.SemaphoreType.DMA for _ in range(U)],
      pltpu.VMEM_SHARED((NP, ncols), jnp.float32),
  ]
  if with_deg:
    scratch.append(pltpu.VMEM((NP,), jnp.float32))

  @functools.partial(pl.kernel, out_type=out_type, mesh=_mesh(),
                     scratch_types=scratch,
                     compiler_params=pltpu.CompilerParams(
                         needs_layout_passes=False))
  def agg(table_hbm, src_hbm, dst_hbm, zero_hbm, *refs):
    if with_deg:
      (out_hbm, deg_hbm, src_r, dst_r, rows, ssems, dsems, gsems, asems, acc,
       deg_local) = refs
    else:
      out_hbm, src_r, dst_r, rows, ssems, dsems, gsems, asems, acc = refs
      deg_local = None
    cid = lax.axis_index("c")
    sid = lax.axis_index("s")
    row0 = pl.multiple_of(sid * rpt, 8)
    base = pl.multiple_of((cid * NS + sid) * ept, 8)
    pltpu.sync_copy(zero_hbm, acc.at[pl.ds(row0, rpt)])
    if with_deg:
      z16 = jnp.zeros((L,), jnp.float32)

      def zbody(i, carry):
        deg_local[pl.ds(pl.multiple_of(i * L, 8), L)] = z16
        return carry

      lax.fori_loop(0, NP // L, zbody, 0)
    plsc.subcore_barrier()
    _edge_loop(table_hbm, src_hbm, dst_hbm, acc, src_r, dst_r, rows,
               ssems, dsems, gsems, asems, base, n_chunks, U, deg_local)
    plsc.subcore_barrier()
    pltpu.sync_copy(acc.at[pl.ds(row0, rpt)], out_hbm.at[cid, pl.ds(row0, rpt)])
    if with_deg:
      pltpu.sync_copy(deg_local, deg_hbm.at[cid * NS + sid])

  return agg


@functools.lru_cache(maxsize=None)
def _make_agg_feat_split():
  ncols = 128
  ept = E // NS
  n_chunks = ept // CH
  rpt = NP // NS

  U = 4

  @functools.partial(
      pl.kernel,
      out_type=jax.ShapeDtypeStruct((NC, NP, ncols), jnp.float32),
      mesh=_mesh(),
      compiler_params=pltpu.CompilerParams(needs_layout_passes=False),
      scratch_types=[
          [pltpu.VMEM((CH,), jnp.int32) for _ in range(U)],
          [pltpu.VMEM((CH,), jnp.int32) for _ in range(U)],
          [pltpu.VMEM((CH, ncols), jnp.float32) for _ in range(U)],
          [pltpu.SemaphoreType.DMA for _ in range(U)],
          [pltpu.SemaphoreType.DMA for _ in range(U)],
          [pltpu.SemaphoreType.DMA for _ in range(U)],
          [pltpu.SemaphoreType.DMA for _ in range(U)],
          pltpu.VMEM_SHARED((NP, ncols), jnp.float32),
      ])
  def agg(tables_hbm, src_hbm, dst_hbm, zero_hbm, out_hbm,
          src_r, dst_r, rows, ssems, dsems, gsems, asems, acc):
    cid = lax.axis_index("c")
    sid = lax.axis_index("s")
    row0 = pl.multiple_of(sid * rpt, 8)
    base = pl.multiple_of(sid * ept, 8)
    pltpu.sync_copy(zero_hbm, acc.at[pl.ds(row0, rpt)])
    plsc.subcore_barrier()

    @pl.when(cid == 0)
    def _():
      _edge_loop(tables_hbm.at[0], src_hbm, dst_hbm, acc, src_r, dst_r, rows,
                 ssems, dsems, gsems, asems, base, n_chunks, U)

    @pl.when(cid == 1)
    def _():
      _edge_loop(tables_hbm.at[1], src_hbm, dst_hbm, acc, src_r, dst_r, rows,
                 ssems, dsems, gsems, asems, base, n_chunks, U)

    plsc.subcore_barrier()
    pltpu.sync_copy(acc.at[pl.ds(row0, rpt)], out_hbm.at[cid, pl.ds(row0, rpt)])

  return agg


_DOT = functools.partial(jnp.dot, preferred_element_type=jnp.float32,
                         precision=jax.lax.Precision.HIGHEST)


def _invdeg_body(dp_ref, inv_ref):
  deg = jnp.sum(dp_ref[...], axis=0)
  inv_ref[...] = 1.0 / jnp.maximum(deg, 1.0)


def _layer1_body(p_ref, d_ref, w_ref, b_ref, h_ref):
  a = (p_ref[0] + p_ref[1]) * d_ref[...]
  h_ref[...] = jnp.maximum(_DOT(a, w_ref[...]) + b_ref[...], 0.0)


def _layer2_body(p_ref, d_ref, w_ref, b_ref, h_ref):
  a = (p_ref[0] + p_ref[1]) * d_ref[...]
  h = jnp.maximum(_DOT(a, w_ref[...]) + b_ref[...], 0.0)
  h_ref[0] = h[:, :128]
  h_ref[1] = h[:, 128:]


def _layer3_body(p_ref, d_ref, w3_ref, b3_ref, wg_ref, bg_ref, out_ref,
                 acc_ref):
  i = pl.program_id(0)

  @pl.when(i == 0)
  def _():
    acc_ref[...] = jnp.zeros_like(acc_ref)

  a = jnp.concatenate([p_ref[0], p_ref[1]], axis=1) * d_ref[...]
  h = jnp.maximum(_DOT(a, w3_ref[...]) + b3_ref[...], 0.0)
  acc_ref[...] += jnp.sum(h, axis=0, keepdims=True)

  @pl.when(i == pl.num_programs(0) - 1)
  def _():
    out_ref[...] = _DOT(acc_ref[...] / N, wg_ref[...]) + bg_ref[...]


def _full(shape):
  return pl.BlockSpec(shape, lambda i: tuple(0 for _ in shape))


_P_SPEC = pl.BlockSpec((2, BM, 128), lambda i: (0, i, 0))
_D_SPEC = pl.BlockSpec((BM, 1), lambda i: (i, 0))


def kernel(x, edge_index, W1, b1, W2, b2, W3, b3, Wg, bg):
  src = edge_index[0]
  dst = edge_index[1]
  z128 = jnp.zeros((NP // NS, 128), jnp.float32)

  parts1, deg_parts = _make_agg_edge_split(True)(x, src, dst, z128)

  inv = pl.pallas_call(
      _invdeg_body,
      grid=(1,),
      in_specs=[_full((NC * NS, DR, 128))],
      out_specs=_full((DR, 128)),
      out_shape=jax.ShapeDtypeStruct((DR, 128), jnp.float32),
  )(deg_parts.reshape(NC * NS, DR, 128))
  deg2col = inv.reshape(NP, 1)

  grid = (N // BM,)
  h1 = pl.pallas_call(
      _layer1_body,
      grid=grid,
      in_specs=[_P_SPEC, _D_SPEC, _full((128, 128)), _full((1, 128))],
      out_specs=pl.BlockSpec((BM, 128), lambda i: (i, 0)),
      out_shape=jax.ShapeDtypeStruct((N, 128), jnp.float32),
  )(parts1, deg2col, W1, b1.reshape(1, 128))

  parts2 = _make_agg_edge_split(False)(h1, src, dst, z128)

  h2p = pl.pallas_call(
      _layer2_body,
      grid=grid,
      in_specs=[_P_SPEC, _D_SPEC, _full((128, 256)), _full((1, 256))],
      out_specs=_P_SPEC,
      out_shape=jax.ShapeDtypeStruct((2, N, 128), jnp.float32),
  )(parts2, deg2col, W2, b2.reshape(1, 256))

  parts3 = _make_agg_feat_split()(h2p, src, dst, z128)

  g = pl.pallas_call(
      _layer3_body,
      grid=grid,
      in_specs=[_P_SPEC, _D_SPEC, _full((256, 512)), _full((1, 512)),
                _full((512, 2)), _full((1, 2))],
      out_specs=pl.BlockSpec((1, 2), lambda i: (0, 0)),
      out_shape=jax.ShapeDtypeStruct((1, 2), jnp.float32),
      scratch_shapes=[pltpu.VMEM((1, 512), jnp.float32)],
  )(parts3, deg2col, W3, b3.reshape(1, 512), Wg, bg.reshape(1, 2))

  return g.reshape(2)


# speedup vs baseline: 11.1865x; 1.0123x over previous
import functools

import jax
import jax.numpy as jnp
from jax import lax
from jax.experimental import pallas as pl
from jax.experimental.pallas import tpu as pltpu
from jax.experimental.pallas import tpu_sc as plsc

N = 10000
NP = 10240
E = 320000
NC = 2
NS = 16
CH = 80
L = 16
DR = NP // 128
BM = 2000


@functools.lru_cache(maxsize=None)
def _mesh():
  return plsc.VectorSubcoreMesh(
      core_axis_name="c", subcore_axis_name="s", num_cores=NC, num_subcores=NS)


def _edge_loop(table_hbm, src_hbm, dst_hbm, acc, src_r, dst_r, rows,
               ssems, dsems, gsems, asems, base, n_chunks, U, deg_local=None):
  ones16 = jnp.ones((L,), jnp.float32)

  def _src_start(c, u):
    off = pl.multiple_of(base + c * CH, 8)
    pltpu.async_copy(src_hbm.at[pl.ds(off, CH)], src_r[u], ssems[u])

  def _dst_start(c, u):
    off = pl.multiple_of(base + c * CH, 8)
    pltpu.async_copy(dst_hbm.at[pl.ds(off, CH)], dst_r[u], dsems[u])

  def _src_wait(u):
    pltpu.make_async_copy(src_hbm.at[pl.ds(0, CH)], src_r[u], ssems[u]).wait()

  def _dst_wait(u):
    pltpu.make_async_copy(dst_hbm.at[pl.ds(0, CH)], dst_r[u], dsems[u]).wait()

  def _gather_start(u):
    pltpu.async_copy(table_hbm.at[src_r[u]], rows[u], gsems[u])

  def _gather_wait(u):
    pltpu.make_async_copy(table_hbm.at[pl.ds(0, CH)], rows[u], gsems[u]).wait()

  def _scatter_wait(u):
    pltpu.make_async_copy(rows[u], acc.at[pl.ds(0, CH)], asems[u]).wait()

  def _hist(u):
    if deg_local is not None:
      for j in range(CH // L):
        d16 = dst_r[u][pl.ds(j * L, L)]
        plsc.addupdate_scatter(deg_local, [d16], ones16)

  for u in range(U):
    _src_start(u, u)
    _dst_start(u, u)
  for u in range(U):
    _src_wait(u)
    _gather_start(u)

  def body(m, carry):
    for u in range(U):
      k = m * U + u
      _gather_wait(u)

      @pl.when(k + U < n_chunks)
      def _():
        _src_start(k + U, u)

      _dst_wait(u)
      pltpu.async_copy(rows[u], acc.at[dst_r[u]], asems[u], add=True)
      _hist(u)

    for u in range(U):
      k2 = m * U + U + u
      _scatter_wait(u)

      @pl.when(k2 < n_chunks)
      def _():
        _dst_start(k2, u)
        _src_wait(u)
        _gather_start(u)

    return carry

  lax.fori_loop(0, n_chunks // U, body, 0)
  for u in range(n_chunks % U):
    _gather_wait(u)
    _dst_wait(u)
    pltpu.sync_copy(rows[u], acc.at[dst_r[u]], add=True)
    _hist(u)


@functools.lru_cache(maxsize=None)
def _make_agg_edge_split(with_deg):
  ncols = 128
  ept = E // (NC * NS)
  n_chunks = ept // CH
  rpt = NP // NS

  U = 3 if with_deg else 4
  out_type = jax.ShapeDtypeStruct((NC, NP, ncols), jnp.float32)
  if with_deg:
    out_type = [out_type, jax.ShapeDtypeStruct((NC * NS, NP), jnp.float32)]
  scratch = [
      [pltpu.VMEM((CH,), jnp.int32) for _ in range(U)],
      [pltpu.VMEM((CH,), jnp.int32) for _ in range(U)],
      [pltpu.VMEM((CH, ncols), jnp.float32) for _ in range(U)],
      [pltpu.SemaphoreType.DMA for _ in range(U)],
      [pltpu.SemaphoreType.DMA for _ in range(U)],
      [pltpu.SemaphoreType.DMA for _ in range(U)],
      [pltpu.SemaphoreType.DMA for _ in range(U)],
      pltpu.VMEM_SHARED((NP, ncols), jnp.float32),
  ]
  if with_deg:
    scratch.append(pltpu.VMEM((NP,), jnp.float32))

  @functools.partial(pl.kernel, out_type=out_type, mesh=_mesh(),
                     scratch_types=scratch,
                     compiler_params=pltpu.CompilerParams(
                         needs_layout_passes=False))
  def agg(table_hbm, src_hbm, dst_hbm, zero_hbm, *refs):
    if with_deg:
      (out_hbm, deg_hbm, src_r, dst_r, rows, ssems, dsems, gsems, asems, acc,
       deg_local) = refs
    else:
      out_hbm, src_r, dst_r, rows, ssems, dsems, gsems, asems, acc = refs
      deg_local = None
    cid = lax.axis_index("c")
    sid = lax.axis_index("s")
    row0 = pl.multiple_of(sid * rpt, 8)
    base = pl.multiple_of((cid * NS + sid) * ept, 8)
    pltpu.sync_copy(zero_hbm, acc.at[pl.ds(row0, rpt)])
    if with_deg:
      z16 = jnp.zeros((L,), jnp.float32)

      def zbody(i, carry):
        deg_local[pl.ds(pl.multiple_of(i * L, 8), L)] = z16
        return carry

      lax.fori_loop(0, NP // L, zbody, 0)
    plsc.subcore_barrier()
    _edge_loop(table_hbm, src_hbm, dst_hbm, acc, src_r, dst_r, rows,
               ssems, dsems, gsems, asems, base, n_chunks, U, deg_local)
    plsc.subcore_barrier()
    pltpu.sync_copy(acc.at[pl.ds(row0, rpt)], out_hbm.at[cid, pl.ds(row0, rpt)])
    if with_deg:
      pltpu.sync_copy(deg_local, deg_hbm.at[cid * NS + sid])

  return agg


@functools.lru_cache(maxsize=None)
def _make_agg_feat_split():
  ncols = 128
  ept = E // NS
  n_chunks = ept // CH
  rpt = NP // NS

  U = 4

  @functools.partial(
      pl.kernel,
      out_type=jax.ShapeDtypeStruct((NC, NP, ncols), jnp.float32),
      mesh=_mesh(),
      compiler_params=pltpu.CompilerParams(needs_layout_passes=False),
      scratch_types=[
          [pltpu.VMEM((CH,), jnp.int32) for _ in range(U)],
          [pltpu.VMEM((CH,), jnp.int32) for _ in range(U)],
          [pltpu.VMEM((CH, ncols), jnp.float32) for _ in range(U)],
          [pltpu.SemaphoreType.DMA for _ in range(U)],
          [pltpu.SemaphoreType.DMA for _ in range(U)],
          [pltpu.SemaphoreType.DMA for _ in range(U)],
          [pltpu.SemaphoreType.DMA for _ in range(U)],
          pltpu.VMEM_SHARED((NP, ncols), jnp.float32),
      ])
  def agg(tables_hbm, src_hbm, dst_hbm, zero_hbm, out_hbm,
          src_r, dst_r, rows, ssems, dsems, gsems, asems, acc):
    cid = lax.axis_index("c")
    sid = lax.axis_index("s")
    row0 = pl.multiple_of(sid * rpt, 8)
    base = pl.multiple_of(sid * ept, 8)
    pltpu.sync_copy(zero_hbm, acc.at[pl.ds(row0, rpt)])
    plsc.subcore_barrier()

    @pl.when(cid == 0)
    def _():
      _edge_loop(tables_hbm.at[0], src_hbm, dst_hbm, acc, src_r, dst_r, rows,
                 ssems, dsems, gsems, asems, base, n_chunks, U)

    @pl.when(cid == 1)
    def _():
      _edge_loop(tables_hbm.at[1], src_hbm, dst_hbm, acc, src_r, dst_r, rows,
                 ssems, dsems, gsems, asems, base, n_chunks, U)

    plsc.subcore_barrier()
    pltpu.sync_copy(acc.at[pl.ds(row0, rpt)], out_hbm.at[cid, pl.ds(row0, rpt)])

  return agg


_DOT = functools.partial(jnp.dot, preferred_element_type=jnp.float32,
                         precision=jax.lax.Precision.HIGHEST)


def _invdeg_body(dp_ref, inv_ref):
  deg = jnp.sum(dp_ref[...], axis=0)
  inv_ref[...] = 1.0 / jnp.maximum(deg, 1.0)


def _layer1_body(p_ref, d_ref, w_ref, b_ref, h_ref):
  a = (p_ref[0] + p_ref[1]) * d_ref[...]
  h_ref[...] = jnp.maximum(_DOT(a, w_ref[...]) + b_ref[...], 0.0)


def _layer2_body(p_ref, d_ref, w_ref, b_ref, h_ref):
  a = (p_ref[0] + p_ref[1]) * d_ref[...]
  h = jnp.maximum(_DOT(a, w_ref[...]) + b_ref[...], 0.0)
  h_ref[0] = h[:, :128]
  h_ref[1] = h[:, 128:]


def _layer3_body(p_ref, d_ref, w3_ref, b3_ref, wg_ref, bg_ref, out_ref,
                 acc_ref):
  i = pl.program_id(0)

  @pl.when(i == 0)
  def _():
    acc_ref[...] = jnp.zeros_like(acc_ref)

  a = jnp.concatenate([p_ref[0], p_ref[1]], axis=1) * d_ref[...]
  h = jnp.maximum(_DOT(a, w3_ref[...]) + b3_ref[...], 0.0)
  acc_ref[...] += jnp.sum(h, axis=0, keepdims=True)

  @pl.when(i == pl.num_programs(0) - 1)
  def _():
    out_ref[...] = _DOT(acc_ref[...] / N, wg_ref[...]) + bg_ref[...]


def _full(shape):
  return pl.BlockSpec(shape, lambda i: tuple(0 for _ in shape))


_P_SPEC = pl.BlockSpec((2, BM, 128), lambda i: (0, i, 0))
_D_SPEC = pl.BlockSpec((BM, 1), lambda i: (i, 0))


def kernel(x, edge_index, W1, b1, W2, b2, W3, b3, Wg, bg):
  src = edge_index[0]
  dst = edge_index[1]
  z128 = jnp.zeros((NP // NS, 128), jnp.float32)

  parts1, deg_parts = _make_agg_edge_split(True)(x, src, dst, z128)

  inv = pl.pallas_call(
      _invdeg_body,
      grid=(1,),
      in_specs=[_full((NC * NS, DR, 128))],
      out_specs=_full((DR, 128)),
      out_shape=jax.ShapeDtypeStruct((DR, 128), jnp.float32),
  )(deg_parts.reshape(NC * NS, DR, 128))
  deg2col = inv.reshape(NP, 1)

  grid = (N // BM,)
  h1 = pl.pallas_call(
      _layer1_body,
      grid=grid,
      in_specs=[_P_SPEC, _D_SPEC, _full((128, 128)), _full((1, 128))],
      out_specs=pl.BlockSpec((BM, 128), lambda i: (i, 0)),
      out_shape=jax.ShapeDtypeStruct((N, 128), jnp.float32),
  )(parts1, deg2col, W1, b1.reshape(1, 128))

  parts2 = _make_agg_edge_split(False)(h1, src, dst, z128)

  h2p = pl.pallas_call(
      _layer2_body,
      grid=grid,
      in_specs=[_P_SPEC, _D_SPEC, _full((128, 256)), _full((1, 256))],
      out_specs=_P_SPEC,
      out_shape=jax.ShapeDtypeStruct((2, N, 128), jnp.float32),
  )(parts2, deg2col, W2, b2.reshape(1, 256))

  parts3 = _make_agg_feat_split()(h2p, src, dst, z128)

  g = pl.pallas_call(
      _layer3_body,
      grid=grid,
      in_specs=[_P_SPEC, _D_SPEC, _full((256, 512)), _full((1, 512)),
                _full((512, 2)), _full((1, 2))],
      out_specs=pl.BlockSpec((1, 2), lambda i: (0, 0)),
      out_shape=jax.ShapeDtypeStruct((1, 2), jnp.float32),
      scratch_shapes=[pltpu.VMEM((1, 512), jnp.float32)],
  )(parts3, deg2col, W3, b3.reshape(1, 512), Wg, bg.reshape(1, 2))

  return g.reshape(2)
